# Initial kernel scaffold; baseline (speedup 1.0000x reference)
import functools

import jax
import jax.numpy as jnp
from jax import lax
from jax.experimental import pallas as pl
from jax.experimental.pallas import tpu as pltpu
from jax.experimental.pallas import tpu_sc as plsc

K = 32
D = 64
N = 65536
B = 8
L = 16
C = 128
QUARTER = N // 4
NCHUNK = QUARTER // C
PUSH_MARGIN = 0.25
PUSH_WEIGHT = 1.0
PULL_WEIGHT = 1.0
REG_WEIGHT = 0.0001

_mesh = plsc.VectorSubcoreMesh(core_axis_name="c", subcore_axis_name="s")


@functools.partial(
    pl.kernel,
    out_type=(
        jax.ShapeDtypeStruct((B, L), jnp.float32),
        jax.ShapeDtypeStruct((32, L), jnp.float32),
    ),
    mesh=_mesh,
    scratch_types=[
        pltpu.VMEM((C, D), jnp.float32),
        pltpu.VMEM((C,), jnp.int32),
        pltpu.VMEM((C,), jnp.int32),
        pltpu.VMEM((C, L), jnp.float32),
        pltpu.VMEM((K, D), jnp.float32),
        pltpu.VMEM((K, L), jnp.float32),
        pltpu.VMEM((L,), jnp.float32),
        pltpu.VMEM_SHARED((4 * K, D), jnp.float32),
        pltpu.VMEM_SHARED((4 * K, L), jnp.float32),
    ],
)
def _sc_loss(emb_hbm, lab_hbm, pushreg_out, pull_out,
             emb_buf, lab_buf, idx_buf, ones_buf, cent_buf, cnt_buf, vec_buf,
             sums_sh, cnts_sh):
    c = lax.axis_index("c")
    s = lax.axis_index("s")
    local_s = s // 4
    quarter = s % 4
    sample = 4 * c + local_s
    wid = c * 16 + s
    base = quarter * QUARTER

    zeros16 = jnp.zeros((L,), jnp.float32)
    ones16 = jnp.ones((L,), jnp.float32)

    def _ones_body(r, _):
        ones_buf[r, :] = ones16
        return 0

    @pl.when(s == 0)
    def _zero_shared():
        def _ze(r, _):
            for jj in range(D // L):
                emb_buf[r, pl.ds(L * jj, L)] = zeros16
            return 0
        lax.fori_loop(0, C, _ze, 0)

        def _zc(r, _):
            cnt_buf[r, :] = zeros16
            return 0
        lax.fori_loop(0, K, _zc, 0)
        pltpu.sync_copy(emb_buf, sums_sh)
        for t in range(4):
            pltpu.sync_copy(cnt_buf, cnts_sh.at[pl.ds(t * K, K)])

    lax.fori_loop(0, C, _ones_body, 0)
    plsc.subcore_barrier()

    row_off = K * local_s

    def _p1_body(k, _):
        off = base + k * C
        pltpu.sync_copy(emb_hbm.at[sample, pl.ds(off, C), :], emb_buf)
        pltpu.sync_copy(lab_hbm.at[sample, pl.ds(off, C)], lab_buf)
        for g in range(C // L):
            lv = lab_buf[pl.ds(L * g, L)]
            idx_buf[pl.ds(L * g, L)] = lv + row_off
        pltpu.sync_copy(emb_buf, sums_sh.at[idx_buf], add=True)
        pltpu.sync_copy(ones_buf, cnts_sh.at[idx_buf], add=True)
        return 0

    lax.fori_loop(0, NCHUNK, _p1_body, 0)
    plsc.subcore_barrier()

    pltpu.sync_copy(sums_sh.at[pl.ds(row_off, K)], cent_buf)
    pltpu.sync_copy(cnts_sh.at[pl.ds(row_off, K)], cnt_buf)

    def _cent_body(r, _):
        cnt = cnt_buf[r, :]
        denom = jnp.maximum(cnt, 1.0)
        valid = cnt > 0.0
        for jj in range(D // L):
            sv = cent_buf[r, pl.ds(L * jj, L)]
            cent_buf[r, pl.ds(L * jj, L)] = jnp.where(valid, sv / denom, 0.0)
        return 0

    lax.fori_loop(0, K, _cent_body, 0)

    @pl.when(quarter == 0)
    def _push_reg():
        def _nv_body(r, acc):
            return acc + jnp.where(cnt_buf[r, :] > 0.0, 1.0, 0.0)
        nv_vec = lax.fori_loop(0, K, _nv_body, zeros16)
        n_valid = jnp.max(nv_vec)

        def _push_i(i, acc_i):
            ci = [cent_buf[i, pl.ds(L * jj, L)] for jj in range(D // L)]
            vi = jnp.max(cnt_buf[i, :])

            def _push_j(j, acc_j):
                dv = zeros16
                for jj in range(D // L):
                    dv = dv + jnp.abs(ci[jj] - cent_buf[j, pl.ds(L * jj, L)])
                dist = jnp.sum(dv)
                vj = jnp.max(cnt_buf[j, :])
                m = (i < j) & (vi > 0.0) & (vj > 0.0)
                h = jnp.maximum(PUSH_MARGIN - dist, 0.0)
                return acc_j + jnp.where(m, h * h, 0.0)

            return lax.fori_loop(0, K, _push_j, acc_i)

        push_sum = lax.fori_loop(0, K, _push_i, jnp.float32(0.0))
        n_comp = n_valid * (n_valid - 1.0) * 0.5
        push_loss = jnp.where(n_valid >= 2.0,
                              push_sum / jnp.maximum(n_comp, 1.0), 0.0)

        def _reg_body(r, acc):
            sq = zeros16
            for jj in range(D // L):
                cv = cent_buf[r, pl.ds(L * jj, L)]
                sq = sq + cv * cv
            return acc + jnp.where(cnt_buf[r, :] > 0.0, sq, 0.0)
        reg_vec = lax.fori_loop(0, K, _reg_body, zeros16)
        reg_loss = jnp.sum(reg_vec) / jnp.maximum(n_valid * float(D), 1.0)

        total = PUSH_WEIGHT * push_loss + REG_WEIGHT * reg_loss
        vec_buf[:] = jnp.full((L,), total, jnp.float32)
        pltpu.sync_copy(vec_buf, pushreg_out.at[sample])

    iota16 = lax.iota(jnp.int32, L)

    def _p2_body(k, pacc):
        off = base + k * C
        pltpu.sync_copy(emb_hbm.at[sample, pl.ds(off, C), :], emb_buf)
        pltpu.sync_copy(lab_hbm.at[sample, pl.ds(off, C)], lab_buf)
        for g in range(C // L):
            lv = lab_buf[pl.ds(L * g, L)]
            pix = iota16 + (L * g)

            def _d_body(d, a):
                dsplat = jnp.full((L,), d, jnp.int32)
                e = plsc.load_gather(emb_buf, [pix, dsplat])
                cv = plsc.load_gather(cent_buf, [lv, dsplat])
                return a + jnp.abs(e - cv)

            pd = lax.fori_loop(0, D, _d_body, zeros16)
            pacc = pacc + pd * pd
        return pacc

    pacc = lax.fori_loop(0, NCHUNK, _p2_body, zeros16)
    vec_buf[:] = pacc
    pltpu.sync_copy(vec_buf, pull_out.at[wid])


def kernel(embeddings, labels):
    lab32 = labels.astype(jnp.int32)
    pushreg, pull = _sc_loss(embeddings, lab32)
    pull_s = pull.reshape(2, 4, 4 * L).sum(axis=-1).reshape(B)
    pull_loss = pull_s / float(N)
    return jnp.mean(pushreg[:, 0] + PULL_WEIGHT * pull_loss)


# speedup vs baseline: 5.0826x; 5.0826x over previous
import functools

import jax
import jax.numpy as jnp
from jax import lax
from jax.experimental import pallas as pl
from jax.experimental.pallas import tpu as pltpu
from jax.experimental.pallas import tpu_sc as plsc

K = 32
D = 64
N = 65536
B = 8
L = 16
C = 128
QUARTER = N // 4
NCHUNK = QUARTER // C
PUSH_MARGIN = 0.25
PUSH_WEIGHT = 1.0
PULL_WEIGHT = 1.0
REG_WEIGHT = 0.0001

_mesh = plsc.VectorSubcoreMesh(core_axis_name="c", subcore_axis_name="s")


@functools.partial(
    pl.kernel,
    out_type=(
        jax.ShapeDtypeStruct((B, L), jnp.float32),
        jax.ShapeDtypeStruct((32, L), jnp.float32),
    ),
    mesh=_mesh,
    compiler_params=pltpu.CompilerParams(needs_layout_passes=False),
    scratch_types=[
        pltpu.VMEM((C, D), jnp.float32),
        pltpu.VMEM((C,), jnp.int32),
        pltpu.VMEM((C,), jnp.int32),
        pltpu.VMEM((C, L), jnp.float32),
        pltpu.VMEM((K, D), jnp.float32),
        pltpu.VMEM((K, L), jnp.float32),
        pltpu.VMEM((L,), jnp.float32),
        pltpu.VMEM_SHARED((4 * K, D), jnp.float32),
        pltpu.VMEM_SHARED((4 * K, L), jnp.float32),
    ],
)
def _sc_loss(emb_hbm, lab_hbm, pushreg_out, pull_out,
             emb_buf, lab_buf, idx_buf, ones_buf, cent_buf, cnt_buf, vec_buf,
             sums_sh, cnts_sh):
    c = lax.axis_index("c")
    s = lax.axis_index("s")
    local_s = s // 4
    quarter = s % 4
    sample = 4 * c + local_s
    wid = c * 16 + s
    base = quarter * QUARTER

    zeros16 = jnp.zeros((L,), jnp.float32)
    ones16 = jnp.ones((L,), jnp.float32)

    def _ones_body(r, _):
        ones_buf[r, :] = ones16
        return 0

    @pl.when(s == 0)
    def _zero_shared():
        def _ze(r, _):
            for jj in range(D // L):
                emb_buf[r, pl.ds(L * jj, L)] = zeros16
            return 0
        lax.fori_loop(0, C, _ze, 0)

        def _zc(r, _):
            cnt_buf[r, :] = zeros16
            return 0
        lax.fori_loop(0, K, _zc, 0)
        pltpu.sync_copy(emb_buf, sums_sh)
        for t in range(4):
            pltpu.sync_copy(cnt_buf, cnts_sh.at[pl.ds(t * K, K)])

    lax.fori_loop(0, C, _ones_body, 0)
    plsc.subcore_barrier()

    row_off = K * local_s

    def _p1_body(k, _):
        off = base + k * C
        pltpu.sync_copy(emb_hbm.at[sample, pl.ds(off, C), :], emb_buf)
        pltpu.sync_copy(lab_hbm.at[sample, pl.ds(off, C)], lab_buf)
        for g in range(C // L):
            lv = lab_buf[pl.ds(L * g, L)]
            idx_buf[pl.ds(L * g, L)] = lv + row_off
        pltpu.sync_copy(emb_buf, sums_sh.at[idx_buf], add=True)
        pltpu.sync_copy(ones_buf, cnts_sh.at[idx_buf], add=True)
        return 0

    lax.fori_loop(0, NCHUNK, _p1_body, 0)
    plsc.subcore_barrier()

    pltpu.sync_copy(sums_sh.at[pl.ds(row_off, K)], cent_buf)
    pltpu.sync_copy(cnts_sh.at[pl.ds(row_off, K)], cnt_buf)

    def _cent_body(r, _):
        cnt = cnt_buf[r, :]
        denom = jnp.maximum(cnt, 1.0)
        valid = cnt > 0.0
        for jj in range(D // L):
            sv = cent_buf[r, pl.ds(L * jj, L)]
            cent_buf[r, pl.ds(L * jj, L)] = jnp.where(valid, sv / denom, 0.0)
        return 0

    lax.fori_loop(0, K, _cent_body, 0)

    iota16 = lax.iota(jnp.int32, L)

    def _hsum_bcast(v):
        for m in (8, 4, 2, 1):
            vec_buf[:] = v
            v = v + plsc.load_gather(vec_buf, [iota16 ^ m])
        return v

    @pl.when(quarter == 0)
    def _push_reg():
        def _nv_body(r, acc):
            return acc + jnp.where(cnt_buf[r, :] > 0.0, ones16, zeros16)
        nv_vec = lax.fori_loop(0, K, _nv_body, zeros16)

        def _push_i(i, acc_i):
            ci = [cent_buf[i, pl.ds(L * jj, L)] for jj in range(D // L)]
            vi = cnt_buf[i, :] > 0.0
            ivec = jnp.full((L,), i, jnp.int32)

            def _push_j(j, acc_j):
                dv = zeros16
                for jj in range(D // L):
                    dv = dv + jnp.abs(ci[jj] - cent_buf[j, pl.ds(L * jj, L)])
                dist = _hsum_bcast(dv)
                vj = cnt_buf[j, :] > 0.0
                m = (ivec < jnp.full((L,), j, jnp.int32)) & vi & vj
                h = jnp.maximum(PUSH_MARGIN - dist, 0.0)
                return acc_j + jnp.where(m, h * h, zeros16)

            return lax.fori_loop(0, K, _push_j, acc_i)

        push_sum = lax.fori_loop(0, K, _push_i, zeros16)
        n_comp = nv_vec * (nv_vec - 1.0) * 0.5
        push_loss = jnp.where(nv_vec >= 2.0,
                              push_sum / jnp.maximum(n_comp, 1.0), zeros16)

        def _reg_body(r, acc):
            sq = zeros16
            for jj in range(D // L):
                cv = cent_buf[r, pl.ds(L * jj, L)]
                sq = sq + cv * cv
            return acc + jnp.where(cnt_buf[r, :] > 0.0, sq, zeros16)
        reg_vec = _hsum_bcast(lax.fori_loop(0, K, _reg_body, zeros16))
        reg_loss = reg_vec / jnp.maximum(nv_vec * float(D), 1.0)

        vec_buf[:] = PUSH_WEIGHT * push_loss + REG_WEIGHT * reg_loss
        pltpu.sync_copy(vec_buf, pushreg_out.at[sample])

    def _p2_body(k, pacc):
        off = base + k * C
        pltpu.sync_copy(emb_hbm.at[sample, pl.ds(off, C), :], emb_buf)
        pltpu.sync_copy(lab_hbm.at[sample, pl.ds(off, C)], lab_buf)
        for g in range(C // L):
            lv = lab_buf[pl.ds(L * g, L)]
            pix = iota16 + (L * g)

            def _d_body(d, a):
                dsplat = jnp.full((L,), d, jnp.int32)
                e = plsc.load_gather(emb_buf, [pix, dsplat])
                cv = plsc.load_gather(cent_buf, [lv, dsplat])
                return a + jnp.abs(e - cv)

            pd = lax.fori_loop(0, D, _d_body, zeros16)
            pacc = pacc + pd * pd
        return pacc

    pacc = lax.fori_loop(0, NCHUNK, _p2_body, zeros16)
    vec_buf[:] = pacc
    pltpu.sync_copy(vec_buf, pull_out.at[wid])


def kernel(embeddings, labels):
    lab32 = labels.astype(jnp.int32)
    pushreg, pull = _sc_loss(embeddings, lab32)
    pull_s = pull.reshape(2, 4, 4 * L).sum(axis=-1).reshape(B)
    pull_loss = pull_s / float(N)
    return jnp.mean(pushreg[:, 0] + PULL_WEIGHT * pull_loss)
